# Initial kernel scaffold; baseline (speedup 1.0000x reference)
import jax
import jax.numpy as jnp
from jax.experimental import pallas as pl


def kernel(input_ids, token_type_ids, word_table, pos_table, type_table, ln_scale, ln_offset):
    raise NotImplementedError("write your pallas kernel here")


# speedup vs baseline: 8.1037x; 8.1037x over previous
import functools

import jax
import jax.numpy as jnp
from jax import lax
from jax.experimental import pallas as pl
from jax.experimental.pallas import tpu as pltpu
from jax.experimental.pallas import tpu_sc as plsc

HIDDEN = 128
EPS = 1e-5

NUM_CORES = 2
NUM_SUBCORES = 16
NUM_WORKERS = NUM_CORES * NUM_SUBCORES
CHUNK = 128


def _sc_gather_body(idx_hbm, table_hbm, out_hbm, idx_v, rows_v, sem):
    c = lax.axis_index("c")
    s = lax.axis_index("s")
    wid = s * NUM_CORES + c
    n_chunks = idx_hbm.shape[1]
    pltpu.sync_copy(idx_hbm.at[wid], idx_v)

    def chunk(j, carry):
        pltpu.async_copy(table_hbm.at[idx_v.at[j]], rows_v, sem).wait()
        pltpu.sync_copy(rows_v, out_hbm.at[wid, j])
        return carry

    lax.fori_loop(0, n_chunks, chunk, 0)


def _sc_gather(word_table, idx_flat):
    n = idx_flat.shape[0]
    assert n % (NUM_WORKERS * CHUNK) == 0
    n_chunks = n // (NUM_WORKERS * CHUNK)
    idx3 = idx_flat.reshape(NUM_WORKERS, n_chunks, CHUNK)
    mesh = plsc.VectorSubcoreMesh(core_axis_name="c", subcore_axis_name="s")
    f = pl.kernel(
        _sc_gather_body,
        out_type=jax.ShapeDtypeStruct((NUM_WORKERS, n_chunks, CHUNK, HIDDEN), jnp.float32),
        mesh=mesh,
        scratch_types=[
            pltpu.VMEM((n_chunks, CHUNK), jnp.int32),
            pltpu.VMEM((CHUNK, HIDDEN), jnp.float32),
            pltpu.SemaphoreType.DMA,
        ],
    )
    return f(idx3, word_table)


def _tc_body(x_ref, tt_ref, pos_ref, type_ref, scale_ref, off_ref, o_ref):
    x = x_ref[...]
    tt = tt_ref[...].reshape(x.shape[0], x.shape[1], 1)
    t0 = type_ref[0][None, None, :]
    t1 = type_ref[1][None, None, :]
    e = x + pos_ref[...][None, :, :] + jnp.where(tt == 0, t0, t1)
    mean = jnp.mean(e, axis=-1, keepdims=True)
    d = e - mean
    var = jnp.mean(d * d, axis=-1, keepdims=True)
    o_ref[...] = d * lax.rsqrt(var + EPS) * scale_ref[...] + off_ref[...]


def _tc_add_ln(gathered, token_type_ids, pos_rows, type_table, ln_scale, ln_offset):
    bsz, seq, hid = gathered.shape
    bb = 8
    grid = (bsz // bb,)
    tt3 = token_type_ids.reshape(bsz, 1, seq)
    return pl.pallas_call(
        _tc_body,
        grid=grid,
        in_specs=[
            pl.BlockSpec((bb, seq, hid), lambda i: (i, 0, 0)),
            pl.BlockSpec((bb, 1, seq), lambda i: (i, 0, 0)),
            pl.BlockSpec((seq, hid), lambda i: (0, 0)),
            pl.BlockSpec((2, hid), lambda i: (0, 0)),
            pl.BlockSpec((1, hid), lambda i: (0, 0)),
            pl.BlockSpec((1, hid), lambda i: (0, 0)),
        ],
        out_specs=pl.BlockSpec((bb, seq, hid), lambda i: (i, 0, 0)),
        out_shape=jax.ShapeDtypeStruct((bsz, seq, hid), jnp.float32),
    )(gathered, tt3, pos_rows, type_table, ln_scale.reshape(1, hid), ln_offset.reshape(1, hid))


def kernel(input_ids, token_type_ids, word_table, pos_table, type_table, ln_scale, ln_offset):
    bsz, seq = input_ids.shape
    idx_flat = input_ids.reshape(-1)
    gathered = _sc_gather(word_table, idx_flat).reshape(bsz, seq, HIDDEN)
    out = _tc_add_ln(gathered, token_type_ids, pos_table[:seq], type_table,
                     ln_scale, ln_offset)
    kl_div = jnp.zeros((), dtype=jnp.float32)
    return (out, kl_div)


# speedup vs baseline: 9.9464x; 1.2274x over previous
import jax
import jax.numpy as jnp
from jax import lax
from jax.experimental import pallas as pl
from jax.experimental.pallas import tpu as pltpu
from jax.experimental.pallas import tpu_sc as plsc

HIDDEN = 128
EPS = 1e-5

NUM_CORES = 2
NUM_SUBCORES = 16
NUM_WORKERS = NUM_CORES * NUM_SUBCORES
SLICES = 4
TC_BLOCK = 8


def _sc_gather_body(idx_hbm, table_hbm, out_hbm, idx_v, rows_v, sem):
    c = lax.axis_index("c")
    s = lax.axis_index("s")
    wid = s * NUM_CORES + c
    n_chunks = idx_hbm.shape[1]
    pltpu.sync_copy(idx_hbm.at[wid], idx_v)

    def chunk(j, carry):
        pltpu.async_copy(table_hbm.at[idx_v.at[j]], rows_v, sem).wait()
        pltpu.sync_copy(rows_v, out_hbm.at[wid, j])
        return carry

    lax.fori_loop(0, n_chunks, chunk, 0)


def _sc_gather(word_table, idx3):
    nw, n_chunks, ch = idx3.shape
    mesh = plsc.VectorSubcoreMesh(core_axis_name="c", subcore_axis_name="s")
    f = pl.kernel(
        _sc_gather_body,
        out_type=jax.ShapeDtypeStruct((nw, n_chunks, ch, HIDDEN), jnp.float32),
        mesh=mesh,
        scratch_types=[
            pltpu.VMEM((n_chunks, ch), jnp.int32),
            pltpu.VMEM((ch, HIDDEN), jnp.float32),
            pltpu.SemaphoreType.DMA,
        ],
    )
    return f(idx3, word_table)


def _ln_math(x_ref, tt_ref, pos_ref, type_ref, scale_ref, off_ref, o_ref):
    x = x_ref[...]
    tt = tt_ref[...].reshape(x.shape[0], x.shape[1], 1)
    t0 = type_ref[0][None, None, :]
    t1 = type_ref[1][None, None, :]
    e = x + pos_ref[...][None, :, :] + jnp.where(tt == 0, t0, t1)
    mean = jnp.mean(e, axis=-1, keepdims=True)
    d = e - mean
    var = jnp.mean(d * d, axis=-1, keepdims=True)
    o_ref[...] = d * lax.rsqrt(var + EPS) * scale_ref[...] + off_ref[...]


def _tc_body(x_ref, tt_ref, pos_ref, type_ref, scale_ref, off_ref, o_ref):
    _ln_math(x_ref, tt_ref, pos_ref, type_ref, scale_ref, off_ref, o_ref)


def _tc_body_alias(x_ref, tt_ref, pos_ref, type_ref, scale_ref, off_ref,
                   big_ref, o_ref):
    del big_ref
    _ln_math(x_ref, tt_ref, pos_ref, type_ref, scale_ref, off_ref, o_ref)


def _tc_slice(g, tt3_k, pos_rows, type_table, scale2, off2, big, k, bsz_total):
    per_b, seq, hid = g.shape
    nblk = per_b // TC_BLOCK
    in_specs = [
        pl.BlockSpec((TC_BLOCK, seq, hid), lambda i: (i, 0, 0)),
        pl.BlockSpec((TC_BLOCK, 1, seq), lambda i: (i, 0, 0)),
        pl.BlockSpec((seq, hid), lambda i: (0, 0)),
        pl.BlockSpec((2, hid), lambda i: (0, 0)),
        pl.BlockSpec((1, hid), lambda i: (0, 0)),
        pl.BlockSpec((1, hid), lambda i: (0, 0)),
    ]
    out_spec = pl.BlockSpec((TC_BLOCK, seq, hid),
                            lambda i, _k=k, _n=nblk: (i + _k * _n, 0, 0))
    args = [g, tt3_k, pos_rows, type_table, scale2, off2]
    if big is None:
        body = _tc_body
        io_alias = {}
    else:
        in_specs.append(pl.BlockSpec(memory_space=pltpu.MemorySpace.HBM))
        args.append(big)
        body = _tc_body_alias
        io_alias = {6: 0}
    return pl.pallas_call(
        body,
        grid=(nblk,),
        in_specs=in_specs,
        out_specs=out_spec,
        out_shape=jax.ShapeDtypeStruct((bsz_total, seq, hid), jnp.float32),
        input_output_aliases=io_alias,
    )(*args)


def _pick_chunk(rows_per_worker):
    for ch in (128, 104, 96, 80, 64, 40, 32, 16, 8):
        if rows_per_worker % ch == 0:
            return ch
    raise ValueError(rows_per_worker)


def kernel(input_ids, token_type_ids, word_table, pos_table, type_table, ln_scale, ln_offset):
    bsz, seq = input_ids.shape
    idx_flat = input_ids.reshape(-1)
    per_b = bsz // SLICES
    rows_per_slice = per_b * seq
    rows_pw = rows_per_slice // NUM_WORKERS
    ch = _pick_chunk(rows_pw)
    pos_rows = pos_table[:seq]
    tt3 = token_type_ids.reshape(bsz, 1, seq)
    scale2 = ln_scale.reshape(1, HIDDEN)
    off2 = ln_offset.reshape(1, HIDDEN)

    big = None
    for k in range(SLICES):
        idx_k = idx_flat[k * rows_per_slice:(k + 1) * rows_per_slice]
        idx_k = idx_k.reshape(NUM_WORKERS, rows_pw // ch, ch)
        g = _sc_gather(word_table, idx_k).reshape(per_b, seq, HIDDEN)
        big = _tc_slice(g, tt3[k * per_b:(k + 1) * per_b], pos_rows,
                        type_table, scale2, off2, big, k, bsz)

    kl_div = jnp.zeros((), dtype=jnp.float32)
    return (big, kl_div)
